# baseline (speedup 1.0000x reference)
import functools

import jax
import jax.numpy as jnp
from jax import lax
from jax.experimental import pallas as pl
from jax.experimental.pallas import tpu as pltpu
from jax.experimental.pallas import tpu_sc as plsc

NTOKEN = 100000
NTOKEN_C = 1000
EMB_DIM = 300
C_EMB_DIM = 64
EMB_PAD = 384
C_EMB_PAD = 128

B0, B1 = 4096, 50
B_TOTAL = B0 * B1
NC, NS = 2, 16
NW = NC * NS
IBLK = B0 // NW


def _make_embed_kernel():
    mesh = plsc.VectorSubcoreMesh(core_axis_name="c", subcore_axis_name="s")

    @functools.partial(
        pl.kernel,
        mesh=mesh,
        out_type=(
            jax.ShapeDtypeStruct((B_TOTAL, EMB_PAD), jnp.float32),
            jax.ShapeDtypeStruct((B_TOTAL, C_EMB_PAD), jnp.float32),
        ),
        scratch_types=[
            pltpu.VMEM((B1, IBLK), jnp.int32),
            pltpu.VMEM((B1, IBLK), jnp.int32),
            pltpu.VMEM((IBLK, EMB_PAD), jnp.float32),
            pltpu.VMEM((IBLK, C_EMB_PAD), jnp.float32),
            pltpu.SemaphoreType.DMA,
        ],
    )
    def embed_kernel(xt_hbm, xct_hbm, emb_hbm, cemb_hbm, out_hbm, outc_hbm,
                     idx_v, idxc_v, rows_v, crows_v, sem):
        wid = lax.axis_index("s") * NC + lax.axis_index("c")
        base = wid * (B1 * IBLK)
        col0 = wid * IBLK
        pltpu.sync_copy(xt_hbm.at[:, pl.ds(col0, IBLK)], idx_v)
        pltpu.sync_copy(xct_hbm.at[:, pl.ds(col0, IBLK)], idxc_v)

        def body(j, carry):
            row0 = base + j * IBLK
            pltpu.async_copy(emb_hbm.at[idx_v.at[j]], rows_v, sem).wait()
            pltpu.sync_copy(rows_v, out_hbm.at[pl.ds(row0, IBLK)])
            pltpu.async_copy(cemb_hbm.at[idxc_v.at[j]], crows_v, sem).wait()
            pltpu.sync_copy(crows_v, outc_hbm.at[pl.ds(row0, IBLK)])
            return carry

        lax.fori_loop(0, B1, body, 0)

    return embed_kernel


_embed = _make_embed_kernel()


def _make_xpose(din_pad, dout):
    def body(i_ref, o_ref):
        o_ref[0] = i_ref[0, 0, :, :dout].T

    return pl.pallas_call(
        body,
        grid=(NW, B1),
        in_specs=[pl.BlockSpec((1, 1, IBLK, din_pad), lambda w, j: (w, j, 0, 0))],
        out_specs=pl.BlockSpec((1, dout, IBLK), lambda w, j: (j, 0, w)),
        out_shape=jax.ShapeDtypeStruct((B1, dout, B0), jnp.float32),
    )


_xpose_emb = _make_xpose(EMB_PAD, EMB_DIM)
_xpose_cemb = _make_xpose(C_EMB_PAD, C_EMB_DIM)


def kernel(x, x_c, emb_W, c_emb_W):
    xt = x.T.astype(jnp.int32)
    xct = x_c.T.astype(jnp.int32)
    emb_p = jnp.pad(emb_W, ((0, 0), (0, EMB_PAD - EMB_DIM)))
    cemb_p = jnp.pad(c_emb_W, ((0, 0), (0, C_EMB_PAD - C_EMB_DIM)))
    out2, outc2 = _embed(xt, xct, emb_p, cemb_p)
    out3 = _xpose_emb(out2.reshape(NW, B1, IBLK, EMB_PAD))
    outc3 = _xpose_cemb(outc2.reshape(NW, B1, IBLK, C_EMB_PAD))
    return (jnp.transpose(out3, (2, 0, 1)), jnp.transpose(outc3, (2, 0, 1)))


# speedup vs baseline: 1.0474x; 1.0474x over previous
import functools

import jax
import jax.numpy as jnp
from jax import lax
from jax.experimental import pallas as pl
from jax.experimental.pallas import tpu as pltpu
from jax.experimental.pallas import tpu_sc as plsc

NTOKEN = 100000
NTOKEN_C = 1000
EMB_DIM = 300
C_EMB_DIM = 64
EMB_PAD = 384
C_EMB_PAD = 128

B0, B1 = 4096, 50
NC, NS = 2, 16
NW = NC * NS
IBLK = B0 // NW
NG = IBLK // 16


def _make_embed_kernel():
    mesh = plsc.VectorSubcoreMesh(core_axis_name="c", subcore_axis_name="s")

    @functools.partial(
        pl.kernel,
        mesh=mesh,
        out_type=(
            jax.ShapeDtypeStruct((B1, EMB_DIM, B0), jnp.float32),
            jax.ShapeDtypeStruct((B1, C_EMB_DIM, B0), jnp.float32),
        ),
        scratch_types=[
            pltpu.VMEM((B1, IBLK), jnp.int32),
            pltpu.VMEM((B1, IBLK), jnp.int32),
            pltpu.VMEM((IBLK, EMB_PAD), jnp.float32),
            pltpu.VMEM((EMB_DIM, IBLK), jnp.float32),
            pltpu.VMEM((IBLK, C_EMB_PAD), jnp.float32),
            pltpu.VMEM((C_EMB_DIM, IBLK), jnp.float32),
            pltpu.SemaphoreType.DMA,
        ],
        compiler_params=pltpu.CompilerParams(needs_layout_passes=False),
    )
    def embed_kernel(xt_hbm, xct_hbm, emb_hbm, cemb_hbm, out_hbm, outc_hbm,
                     idx_v, idxc_v, rows_v, rowst_v, crows_v, crowst_v, sem):
        wid = lax.axis_index("s") * NC + lax.axis_index("c")
        col0 = wid * IBLK
        pltpu.sync_copy(xt_hbm.at[:, pl.ds(col0, IBLK)], idx_v)
        pltpu.sync_copy(xct_hbm.at[:, pl.ds(col0, IBLK)], idxc_v)

        tok16 = [lax.iota(jnp.int32, 16) + 16 * g for g in range(NG)]

        def xpose(src, dst, d_hi, unroll):
            def tbody(t, carry):
                for u in range(unroll):
                    d = t * unroll + u
                    dvec = jnp.full((16,), d, jnp.int32)
                    for g in range(NG):
                        vals = plsc.load_gather(src, [tok16[g], dvec])
                        dst[d, pl.ds(16 * g, 16)] = vals
                return carry

            lax.fori_loop(0, d_hi // unroll, tbody, 0)

        def body(j, carry):
            pltpu.async_copy(emb_hbm.at[idx_v.at[j]], rows_v, sem).wait()
            xpose(rows_v, rowst_v, EMB_DIM, 4)
            pltpu.sync_copy(rowst_v, out_hbm.at[j, :, pl.ds(col0, IBLK)])
            pltpu.async_copy(cemb_hbm.at[idxc_v.at[j]], crows_v, sem).wait()
            xpose(crows_v, crowst_v, C_EMB_DIM, 4)
            pltpu.sync_copy(crowst_v, outc_hbm.at[j, :, pl.ds(col0, IBLK)])
            return carry

        lax.fori_loop(0, B1, body, 0)

    return embed_kernel


_embed = _make_embed_kernel()


def kernel(x, x_c, emb_W, c_emb_W):
    xt = x.T.astype(jnp.int32)
    xct = x_c.T.astype(jnp.int32)
    emb_p = jnp.pad(emb_W, ((0, 0), (0, EMB_PAD - EMB_DIM)))
    cemb_p = jnp.pad(c_emb_W, ((0, 0), (0, C_EMB_PAD - C_EMB_DIM)))
    out3, outc3 = _embed(xt, xct, emb_p, cemb_p)
    return (jnp.transpose(out3, (2, 0, 1)), jnp.transpose(outc3, (2, 0, 1)))


# speedup vs baseline: 1.6282x; 1.5545x over previous
import functools

import jax
import jax.numpy as jnp
from jax import lax
from jax.experimental import pallas as pl
from jax.experimental.pallas import tpu as pltpu
from jax.experimental.pallas import tpu_sc as plsc

NTOKEN = 100000
NTOKEN_C = 1000
EMB_DIM = 300
C_EMB_DIM = 64
EMB_PAD = 384
C_EMB_PAD = 128

B0, B1 = 4096, 50
NC, NS = 2, 16
NW = NC * NS
IBLK = B0 // NW
NG = IBLK // 16


def _make_embed_kernel():
    mesh = plsc.VectorSubcoreMesh(core_axis_name="c", subcore_axis_name="s")

    @functools.partial(
        pl.kernel,
        mesh=mesh,
        out_type=(
            jax.ShapeDtypeStruct((B1, EMB_DIM, B0), jnp.float32),
            jax.ShapeDtypeStruct((B1, C_EMB_DIM, B0), jnp.float32),
        ),
        scratch_types=[
            pltpu.VMEM((B1, IBLK), jnp.int32),
            pltpu.VMEM((B1, IBLK), jnp.int32),
            pltpu.VMEM((IBLK, EMB_PAD), jnp.float32),
            pltpu.VMEM((EMB_DIM, IBLK), jnp.float32),
            pltpu.VMEM((IBLK, C_EMB_PAD), jnp.float32),
            pltpu.VMEM((C_EMB_DIM, IBLK), jnp.float32),
            pltpu.SemaphoreType.DMA,
            pltpu.SemaphoreType.DMA,
            pltpu.SemaphoreType.DMA,
            pltpu.SemaphoreType.DMA,
        ],
        compiler_params=pltpu.CompilerParams(needs_layout_passes=False),
    )
    def embed_kernel(xt_hbm, xct_hbm, emb_hbm, cemb_hbm, out_hbm, outc_hbm,
                     idx_v, idxc_v, rows_v, rowst_v, crows_v, crowst_v,
                     gsem, gsem2, wsem, wsem2):
        wid = lax.axis_index("s") * NC + lax.axis_index("c")
        col0 = wid * IBLK
        pltpu.sync_copy(xt_hbm.at[:, pl.ds(col0, IBLK)], idx_v)
        pltpu.sync_copy(xct_hbm.at[:, pl.ds(col0, IBLK)], idxc_v)

        tok16 = [lax.iota(jnp.int32, 16) + 16 * g for g in range(NG)]

        def xpose(src, dst, d_hi):
            @plsc.parallel_loop(0, d_hi, 1, unroll=8)
            def _(d):
                dvec = jnp.full((16,), d, jnp.int32)
                for g in range(NG):
                    vals = plsc.load_gather(src, [tok16[g], dvec])
                    dst[d, pl.ds(16 * g, 16)] = vals

        def body(j, carry):
            out_slc = out_hbm.at[j, :, pl.ds(col0, IBLK)]
            outc_slc = outc_hbm.at[j, :, pl.ds(col0, IBLK)]
            pltpu.async_copy(emb_hbm.at[idx_v.at[j]], rows_v, gsem)
            pltpu.async_copy(cemb_hbm.at[idxc_v.at[j]], crows_v, gsem2)

            @pl.when(j > 0)
            def _():
                pltpu.make_async_copy(rowst_v, out_slc, wsem).wait()
                pltpu.make_async_copy(crowst_v, outc_slc, wsem2).wait()

            pltpu.make_async_copy(emb_hbm.at[idx_v.at[j]], rows_v, gsem).wait()
            xpose(rows_v, rowst_v, EMB_DIM)
            pltpu.async_copy(rowst_v, out_slc, wsem)
            pltpu.make_async_copy(cemb_hbm.at[idxc_v.at[j]], crows_v,
                                  gsem2).wait()
            xpose(crows_v, crowst_v, C_EMB_DIM)
            pltpu.async_copy(crowst_v, outc_slc, wsem2)
            return carry

        lax.fori_loop(0, B1, body, 0)
        pltpu.make_async_copy(
            rowst_v, out_hbm.at[0, :, pl.ds(col0, IBLK)], wsem).wait()
        pltpu.make_async_copy(
            crowst_v, outc_hbm.at[0, :, pl.ds(col0, IBLK)], wsem2).wait()

    return embed_kernel


_embed = _make_embed_kernel()


def kernel(x, x_c, emb_W, c_emb_W):
    xt = x.T.astype(jnp.int32)
    xct = x_c.T.astype(jnp.int32)
    emb_p = jnp.pad(emb_W, ((0, 0), (0, EMB_PAD - EMB_DIM)))
    cemb_p = jnp.pad(c_emb_W, ((0, 0), (0, C_EMB_PAD - C_EMB_DIM)))
    out3, outc3 = _embed(xt, xct, emb_p, cemb_p)
    return (jnp.transpose(out3, (2, 0, 1)), jnp.transpose(outc3, (2, 0, 1)))
